# baseline (speedup 1.0000x reference)
import functools

import jax
import jax.numpy as jnp
from jax import lax
from jax.experimental import pallas as pl
from jax.experimental.pallas import tpu as pltpu
from jax.experimental.pallas import tpu_sc as plsc

N_NODES = 10000
N_PAD = 10240
N_FEAT = 128
N_HID = 128
N_CLASS = 64
N_EDGES = 320000
N_EDGES_LOOP = N_EDGES + N_NODES

CHUNK = 128
N_WORKERS = 32
CHUNKS_PER_W = 82
EDGES_PER_W = CHUNKS_PER_W * CHUNK
E_PAD = N_WORKERS * EDGES_PER_W
STRIPE = N_PAD // 16
DEG_W = 128

_mesh = plsc.VectorSubcoreMesh(core_axis_name="c", subcore_axis_name="s")


@functools.partial(
    pl.kernel,
    mesh=_mesh,
    out_type=jax.ShapeDtypeStruct((2, N_PAD, DEG_W), jnp.float32),
    scratch_types=[
        pltpu.VMEM((CHUNK,), jnp.int32),
        pltpu.VMEM((CHUNK, DEG_W), jnp.float32),
        pltpu.VMEM((16, DEG_W), jnp.float32),
        pltpu.VMEM_SHARED((N_PAD, DEG_W), jnp.float32),
    ],
)
def _deg_kernel(dst_hbm, out_hbm, dst_v, ones_v, zrow_v, d_sh):
    c = lax.axis_index("c")
    s = lax.axis_index("s")

    zero16 = jnp.zeros((16,), jnp.float32)
    one16 = jnp.ones((16,), jnp.float32)
    for r in range(16):
        for q in range(DEG_W // 16):
            zrow_v[r, pl.ds(q * 16, 16)] = zero16
    for r in range(CHUNK):
        for q in range(DEG_W // 16):
            ones_v[r, pl.ds(q * 16, 16)] = one16

    base_row = s * STRIPE
    def zero_body(i, carry):
        pltpu.sync_copy(zrow_v, d_sh.at[pl.ds(base_row + i * 16, 16)])
        return carry
    lax.fori_loop(0, STRIPE // 16, zero_body, 0)
    plsc.subcore_barrier()

    wid = c * 16 + s
    ebase = wid * EDGES_PER_W
    def edge_body(j, carry):
        pltpu.sync_copy(dst_hbm.at[pl.ds(ebase + j * CHUNK, CHUNK)], dst_v)
        pltpu.sync_copy(ones_v, d_sh.at[dst_v], add=True)
        return carry
    lax.fori_loop(0, CHUNKS_PER_W, edge_body, 0)
    plsc.subcore_barrier()

    pltpu.sync_copy(d_sh.at[pl.ds(base_row, STRIPE)],
                    out_hbm.at[c, pl.ds(base_row, STRIPE)])


@functools.partial(
    pl.kernel,
    mesh=_mesh,
    out_type=jax.ShapeDtypeStruct((2, N_PAD, N_FEAT), jnp.float32),
    scratch_types=[
        pltpu.VMEM((CHUNK,), jnp.int32),
        pltpu.VMEM((CHUNK,), jnp.int32),
        pltpu.VMEM((CHUNK, N_FEAT), jnp.float32),
        pltpu.VMEM((16, N_FEAT), jnp.float32),
        pltpu.VMEM_SHARED((N_PAD, N_FEAT), jnp.float32),
        pltpu.SemaphoreType.DMA,
    ],
)
def _seg_sum_kernel(g_hbm, src_hbm, dst_hbm, out_hbm,
                    src_v, dst_v, rows_v, zrow_v, s_sh, sem):
    c = lax.axis_index("c")
    s = lax.axis_index("s")

    zero16 = jnp.zeros((16,), jnp.float32)
    for r in range(16):
        for q in range(N_FEAT // 16):
            zrow_v[r, pl.ds(q * 16, 16)] = zero16

    base_row = s * STRIPE
    def zero_body(i, carry):
        pltpu.sync_copy(zrow_v, s_sh.at[pl.ds(base_row + i * 16, 16)])
        return carry
    lax.fori_loop(0, STRIPE // 16, zero_body, 0)
    plsc.subcore_barrier()

    wid = c * 16 + s
    ebase = wid * EDGES_PER_W
    def edge_body(j, carry):
        off = ebase + j * CHUNK
        pltpu.sync_copy(src_hbm.at[pl.ds(off, CHUNK)], src_v)
        pltpu.sync_copy(dst_hbm.at[pl.ds(off, CHUNK)], dst_v)
        pltpu.async_copy(g_hbm.at[src_v], rows_v, sem).wait()
        pltpu.sync_copy(rows_v, s_sh.at[dst_v], add=True)
        return carry
    lax.fori_loop(0, CHUNKS_PER_W, edge_body, 0)
    plsc.subcore_barrier()

    pltpu.sync_copy(s_sh.at[pl.ds(base_row, STRIPE)],
                    out_hbm.at[c, pl.ds(base_row, STRIPE)])


ROWS = 1024
GRID = N_PAD // ROWS


def _dinv_from(dp_ref):
    deg = dp_ref[0, :, 0:1] + dp_ref[1, :, 0:1]
    return jnp.where(deg > 0.0, lax.rsqrt(deg), 0.0)


def _tc_a_body(x_ref, w_ref, dp_ref, o_ref):
    dinv = _dinv_from(dp_ref)
    h = jnp.dot(x_ref[...], w_ref[...], preferred_element_type=jnp.float32)
    o_ref[...] = h * dinv


def _tc_b_body(sp_ref, dp_ref, b_ref, w_ref, o_ref):
    dinv = _dinv_from(dp_ref)
    sm = sp_ref[0] + sp_ref[1]
    z = jnp.maximum(sm * dinv + b_ref[...], 0.0)
    h = jnp.dot(z, w_ref[...], preferred_element_type=jnp.float32)
    o_ref[...] = h * dinv


def _tc_c_body(sp_ref, dp_ref, b2_ref, w_ref, b3_ref, o_ref):
    dinv = _dinv_from(dp_ref)
    sm = sp_ref[0] + sp_ref[1]
    z = jnp.maximum(sm * dinv + b2_ref[...], 0.0)
    o_ref[...] = (
        jnp.dot(z, w_ref[...], preferred_element_type=jnp.float32) + b3_ref[...]
    )


def _tc_a(xp, W1, degp):
    return pl.pallas_call(
        _tc_a_body,
        grid=(GRID,),
        in_specs=[
            pl.BlockSpec((ROWS, N_FEAT), lambda i: (i, 0)),
            pl.BlockSpec((N_FEAT, N_HID), lambda i: (0, 0)),
            pl.BlockSpec((2, ROWS, DEG_W), lambda i: (0, i, 0)),
        ],
        out_specs=pl.BlockSpec((ROWS, N_HID), lambda i: (i, 0)),
        out_shape=jax.ShapeDtypeStruct((N_PAD, N_HID), jnp.float32),
    )(xp, W1, degp)


def _tc_b(sp, degp, b1, W2):
    return pl.pallas_call(
        _tc_b_body,
        grid=(GRID,),
        in_specs=[
            pl.BlockSpec((2, ROWS, N_HID), lambda i: (0, i, 0)),
            pl.BlockSpec((2, ROWS, DEG_W), lambda i: (0, i, 0)),
            pl.BlockSpec((1, N_HID), lambda i: (0, 0)),
            pl.BlockSpec((N_HID, N_HID), lambda i: (0, 0)),
        ],
        out_specs=pl.BlockSpec((ROWS, N_HID), lambda i: (i, 0)),
        out_shape=jax.ShapeDtypeStruct((N_PAD, N_HID), jnp.float32),
    )(sp, degp, b1, W2)


def _tc_c(sp, degp, b2, W3, b3):
    return pl.pallas_call(
        _tc_c_body,
        grid=(GRID,),
        in_specs=[
            pl.BlockSpec((2, ROWS, N_HID), lambda i: (0, i, 0)),
            pl.BlockSpec((2, ROWS, DEG_W), lambda i: (0, i, 0)),
            pl.BlockSpec((1, N_HID), lambda i: (0, 0)),
            pl.BlockSpec((N_HID, N_CLASS), lambda i: (0, 0)),
            pl.BlockSpec((1, N_CLASS), lambda i: (0, 0)),
        ],
        out_specs=pl.BlockSpec((ROWS, N_CLASS), lambda i: (i, 0)),
        out_shape=jax.ShapeDtypeStruct((N_PAD, N_CLASS), jnp.float32),
    )(sp, degp, b2, W3, b3)


def kernel(x, edge_index, W1, b1, W2, b2, W3, b3):
    ei = edge_index.astype(jnp.int32)
    loop_idx = jnp.arange(N_NODES, dtype=jnp.int32)
    pad = jnp.full((E_PAD - N_EDGES_LOOP,), N_PAD - 1, jnp.int32)
    src = jnp.concatenate([ei[0], loop_idx, pad])
    dst = jnp.concatenate([ei[1], loop_idx, pad])
    xp = jnp.pad(x, ((0, N_PAD - N_NODES), (0, 0)))

    degp = _deg_kernel(dst)
    g1 = _tc_a(xp, W1, degp)
    s1 = _seg_sum_kernel(g1, src, dst)
    g2 = _tc_b(s1, degp, b1.reshape(1, -1), W2)
    s2 = _seg_sum_kernel(g2, src, dst)
    out = _tc_c(s2, degp, b2.reshape(1, -1), W3, b3.reshape(1, -1))
    return out[:N_NODES]


# speedup vs baseline: 1.5091x; 1.5091x over previous
import functools

import jax
import jax.numpy as jnp
from jax import lax
from jax.experimental import pallas as pl
from jax.experimental.pallas import tpu as pltpu
from jax.experimental.pallas import tpu_sc as plsc

N_NODES = 10000
N_PAD = 10240
N_FEAT = 128
N_HID = 128
N_CLASS = 64
N_EDGES = 320000
N_EDGES_LOOP = N_EDGES + N_NODES

CHUNK = 128
N_WORKERS = 32
CHUNKS_PER_W = 81
EDGES_PER_W = CHUNKS_PER_W * CHUNK
E_PAD = N_WORKERS * EDGES_PER_W
STRIPE = N_PAD // 16
DEG_W = 128

_mesh = plsc.VectorSubcoreMesh(core_axis_name="c", subcore_axis_name="s")


@functools.partial(
    pl.kernel,
    mesh=_mesh,
    out_type=jax.ShapeDtypeStruct((2, N_PAD, DEG_W), jnp.float32),
    scratch_types=[
        pltpu.VMEM((CHUNK,), jnp.int32),
        pltpu.VMEM((CHUNK, DEG_W), jnp.float32),
        pltpu.VMEM((16, DEG_W), jnp.float32),
        pltpu.VMEM_SHARED((N_PAD, DEG_W), jnp.float32),
    ],
)
def _deg_kernel(dst_hbm, out_hbm, dst_v, ones_v, zrow_v, d_sh):
    c = lax.axis_index("c")
    s = lax.axis_index("s")

    zero16 = jnp.zeros((16,), jnp.float32)
    one16 = jnp.ones((16,), jnp.float32)
    for r in range(16):
        for q in range(DEG_W // 16):
            zrow_v[r, pl.ds(q * 16, 16)] = zero16
    for r in range(CHUNK):
        for q in range(DEG_W // 16):
            ones_v[r, pl.ds(q * 16, 16)] = one16

    base_row = s * STRIPE
    def zero_body(i, carry):
        pltpu.sync_copy(zrow_v, d_sh.at[pl.ds(base_row + i * 16, 16)])
        return carry
    lax.fori_loop(0, STRIPE // 16, zero_body, 0)
    plsc.subcore_barrier()

    wid = c * 16 + s
    ebase = wid * EDGES_PER_W
    def edge_body(j, carry):
        pltpu.sync_copy(dst_hbm.at[pl.ds(ebase + j * CHUNK, CHUNK)], dst_v)
        pltpu.sync_copy(ones_v, d_sh.at[dst_v], add=True)
        return carry
    lax.fori_loop(0, CHUNKS_PER_W, edge_body, 0)
    plsc.subcore_barrier()

    pltpu.sync_copy(d_sh.at[pl.ds(base_row, STRIPE)],
                    out_hbm.at[c, pl.ds(base_row, STRIPE)])


@functools.partial(
    pl.kernel,
    mesh=_mesh,
    out_type=jax.ShapeDtypeStruct((2, N_PAD, N_FEAT), jnp.float32),
    scratch_types=[
        pltpu.VMEM((CHUNK,), jnp.int32),
        pltpu.VMEM((CHUNK,), jnp.int32),
        pltpu.VMEM((CHUNK, N_FEAT), jnp.float32),
        pltpu.VMEM((16, N_FEAT), jnp.float32),
        pltpu.VMEM_SHARED((N_PAD, N_FEAT), jnp.float32),
        pltpu.SemaphoreType.DMA,
    ],
)
def _seg_sum_kernel(g_hbm, src_hbm, dst_hbm, out_hbm,
                    src_v, dst_v, rows_v, zrow_v, s_sh, sem):
    c = lax.axis_index("c")
    s = lax.axis_index("s")

    zero16 = jnp.zeros((16,), jnp.float32)
    for r in range(16):
        for q in range(N_FEAT // 16):
            zrow_v[r, pl.ds(q * 16, 16)] = zero16

    base_row = s * STRIPE
    def zero_body(i, carry):
        pltpu.sync_copy(zrow_v, s_sh.at[pl.ds(base_row + i * 16, 16)])
        return carry
    lax.fori_loop(0, STRIPE // 16, zero_body, 0)
    plsc.subcore_barrier()

    wid = c * 16 + s
    ebase = wid * EDGES_PER_W
    def edge_body(j, carry):
        off = ebase + j * CHUNK
        pltpu.sync_copy(src_hbm.at[pl.ds(off, CHUNK)], src_v)
        pltpu.sync_copy(dst_hbm.at[pl.ds(off, CHUNK)], dst_v)
        pltpu.async_copy(g_hbm.at[src_v], rows_v, sem).wait()
        pltpu.sync_copy(rows_v, s_sh.at[dst_v], add=True)
        return carry
    lax.fori_loop(0, CHUNKS_PER_W, edge_body, 0)
    plsc.subcore_barrier()

    pltpu.sync_copy(s_sh.at[pl.ds(base_row, STRIPE)],
                    out_hbm.at[c, pl.ds(base_row, STRIPE)])


ROWS = 1024
GRID = N_PAD // ROWS


def _dinv_from(dp_ref):
    deg = dp_ref[0, :, 0:1] + dp_ref[1, :, 0:1]
    return jnp.where(deg > 0.0, lax.rsqrt(deg), 0.0)


def _tc_a_body(x_ref, w_ref, dp_ref, o_ref):
    dinv = _dinv_from(dp_ref)
    h = jnp.dot(x_ref[...], w_ref[...], preferred_element_type=jnp.float32)
    o_ref[...] = h * dinv


def _tc_b_body(sp_ref, dp_ref, b_ref, w_ref, o_ref):
    dinv = _dinv_from(dp_ref)
    sm = sp_ref[0] + sp_ref[1]
    z = jnp.maximum(sm * dinv + b_ref[...], 0.0)
    h = jnp.dot(z, w_ref[...], preferred_element_type=jnp.float32)
    o_ref[...] = h * dinv


def _tc_c_body(sp_ref, dp_ref, b2_ref, w_ref, b3_ref, o_ref):
    dinv = _dinv_from(dp_ref)
    sm = sp_ref[0] + sp_ref[1]
    z = jnp.maximum(sm * dinv + b2_ref[...], 0.0)
    o_ref[...] = (
        jnp.dot(z, w_ref[...], preferred_element_type=jnp.float32) + b3_ref[...]
    )


def _tc_a(xp, W1, degp):
    return pl.pallas_call(
        _tc_a_body,
        grid=(GRID,),
        in_specs=[
            pl.BlockSpec((ROWS, N_FEAT), lambda i: (i, 0)),
            pl.BlockSpec((N_FEAT, N_HID), lambda i: (0, 0)),
            pl.BlockSpec((2, ROWS, DEG_W), lambda i: (0, i, 0)),
        ],
        out_specs=pl.BlockSpec((ROWS, N_HID), lambda i: (i, 0)),
        out_shape=jax.ShapeDtypeStruct((N_PAD, N_HID), jnp.float32),
    )(xp, W1, degp)


def _tc_b(sp, degp, b1, W2):
    return pl.pallas_call(
        _tc_b_body,
        grid=(GRID,),
        in_specs=[
            pl.BlockSpec((2, ROWS, N_HID), lambda i: (0, i, 0)),
            pl.BlockSpec((2, ROWS, DEG_W), lambda i: (0, i, 0)),
            pl.BlockSpec((1, N_HID), lambda i: (0, 0)),
            pl.BlockSpec((N_HID, N_HID), lambda i: (0, 0)),
        ],
        out_specs=pl.BlockSpec((ROWS, N_HID), lambda i: (i, 0)),
        out_shape=jax.ShapeDtypeStruct((N_PAD, N_HID), jnp.float32),
    )(sp, degp, b1, W2)


def _tc_c(sp, degp, b2, W3, b3):
    return pl.pallas_call(
        _tc_c_body,
        grid=(GRID,),
        in_specs=[
            pl.BlockSpec((2, ROWS, N_HID), lambda i: (0, i, 0)),
            pl.BlockSpec((2, ROWS, DEG_W), lambda i: (0, i, 0)),
            pl.BlockSpec((1, N_HID), lambda i: (0, 0)),
            pl.BlockSpec((N_HID, N_CLASS), lambda i: (0, 0)),
            pl.BlockSpec((1, N_CLASS), lambda i: (0, 0)),
        ],
        out_specs=pl.BlockSpec((ROWS, N_CLASS), lambda i: (i, 0)),
        out_shape=jax.ShapeDtypeStruct((N_PAD, N_CLASS), jnp.float32),
    )(sp, degp, b2, W3, b3)


def kernel(x, edge_index, W1, b1, W2, b2, W3, b3):
    ei = edge_index.astype(jnp.int32)
    loop_idx = jnp.arange(N_NODES, dtype=jnp.int32)
    pad = jnp.full((E_PAD - N_EDGES_LOOP,), N_PAD - 1, jnp.int32)
    src = jnp.concatenate([ei[0], loop_idx, pad])
    dst = jnp.concatenate([ei[1], loop_idx, pad])
    xp = jnp.pad(x, ((0, N_PAD - N_NODES), (0, 0)))

    degp = _deg_kernel(dst)
    g1 = _tc_a(xp, W1, degp)
    s1 = _seg_sum_kernel(g1, src, dst)
    g2 = _tc_b(s1, degp, b1.reshape(1, -1), W2)
    s2 = _seg_sum_kernel(g2, src, dst)
    out = _tc_c(s2, degp, b2.reshape(1, -1), W3, b3.reshape(1, -1))
    return out[:N_NODES]


# speedup vs baseline: 1.5217x; 1.0083x over previous
import functools

import jax
import jax.numpy as jnp
from jax import lax
from jax.experimental import pallas as pl
from jax.experimental.pallas import tpu as pltpu
from jax.experimental.pallas import tpu_sc as plsc

N_NODES = 10000
N_PAD = 10240
N_FEAT = 128
N_HID = 128
N_CLASS = 64
N_EDGES = 320000
N_EDGES_LOOP = N_EDGES + N_NODES

CHUNK = 128
N_WORKERS = 32
CHUNKS_PER_W = 81
EDGES_PER_W = CHUNKS_PER_W * CHUNK
E_PAD = N_WORKERS * EDGES_PER_W
STRIPE = N_PAD // 16
DEG_W = 128

_mesh = plsc.VectorSubcoreMesh(core_axis_name="c", subcore_axis_name="s")


@functools.partial(
    pl.kernel,
    mesh=_mesh,
    out_type=jax.ShapeDtypeStruct((2, N_PAD, DEG_W), jnp.float32),
    scratch_types=[
        pltpu.VMEM((CHUNK,), jnp.int32),
        pltpu.VMEM((CHUNK, DEG_W), jnp.float32),
        pltpu.VMEM((16, DEG_W), jnp.float32),
        pltpu.VMEM_SHARED((N_PAD, DEG_W), jnp.float32),
    ],
)
def _deg_kernel(dst_hbm, out_hbm, dst_v, ones_v, zrow_v, d_sh):
    c = lax.axis_index("c")
    s = lax.axis_index("s")

    zero16 = jnp.zeros((16,), jnp.float32)
    one16 = jnp.ones((16,), jnp.float32)
    for r in range(16):
        for q in range(DEG_W // 16):
            zrow_v[r, pl.ds(q * 16, 16)] = zero16
    for r in range(CHUNK):
        for q in range(DEG_W // 16):
            ones_v[r, pl.ds(q * 16, 16)] = one16

    base_row = s * STRIPE
    def zero_body(i, carry):
        pltpu.sync_copy(zrow_v, d_sh.at[pl.ds(base_row + i * 16, 16)])
        return carry
    lax.fori_loop(0, STRIPE // 16, zero_body, 0)
    plsc.subcore_barrier()

    wid = c * 16 + s
    def edge_body(j, carry):
        off = (j * N_WORKERS + wid) * CHUNK
        pltpu.sync_copy(dst_hbm.at[pl.ds(off, CHUNK)], dst_v)
        pltpu.sync_copy(ones_v, d_sh.at[dst_v], add=True)
        return carry
    lax.fori_loop(0, CHUNKS_PER_W, edge_body, 0)
    plsc.subcore_barrier()

    pltpu.sync_copy(d_sh.at[pl.ds(base_row, STRIPE)],
                    out_hbm.at[c, pl.ds(base_row, STRIPE)])


@functools.partial(
    pl.kernel,
    mesh=_mesh,
    out_type=jax.ShapeDtypeStruct((2, N_PAD, N_FEAT), jnp.float32),
    scratch_types=[
        pltpu.VMEM((CHUNK,), jnp.int32),
        pltpu.VMEM((CHUNK,), jnp.int32),
        pltpu.VMEM((CHUNK, N_FEAT), jnp.float32),
        pltpu.VMEM((16, N_FEAT), jnp.float32),
        pltpu.VMEM_SHARED((N_PAD, N_FEAT), jnp.float32),
        pltpu.SemaphoreType.DMA,
    ],
)
def _seg_sum_kernel(g_hbm, src_hbm, dst_hbm, out_hbm,
                    src_v, dst_v, rows_v, zrow_v, s_sh, sem):
    c = lax.axis_index("c")
    s = lax.axis_index("s")

    zero16 = jnp.zeros((16,), jnp.float32)
    for r in range(16):
        for q in range(N_FEAT // 16):
            zrow_v[r, pl.ds(q * 16, 16)] = zero16

    base_row = s * STRIPE
    def zero_body(i, carry):
        pltpu.sync_copy(zrow_v, s_sh.at[pl.ds(base_row + i * 16, 16)])
        return carry
    lax.fori_loop(0, STRIPE // 16, zero_body, 0)
    plsc.subcore_barrier()

    wid = c * 16 + s
    def edge_body(j, carry):
        off = (j * N_WORKERS + wid) * CHUNK
        pltpu.sync_copy(src_hbm.at[pl.ds(off, CHUNK)], src_v)
        pltpu.sync_copy(dst_hbm.at[pl.ds(off, CHUNK)], dst_v)
        pltpu.async_copy(g_hbm.at[src_v], rows_v, sem).wait()
        pltpu.sync_copy(rows_v, s_sh.at[dst_v], add=True)
        return carry
    lax.fori_loop(0, CHUNKS_PER_W, edge_body, 0)
    plsc.subcore_barrier()

    pltpu.sync_copy(s_sh.at[pl.ds(base_row, STRIPE)],
                    out_hbm.at[c, pl.ds(base_row, STRIPE)])


ROWS = 1024
GRID = N_PAD // ROWS


def _dinv_from(dp_ref):
    deg = dp_ref[0, :, 0:1] + dp_ref[1, :, 0:1]
    return jnp.where(deg > 0.0, lax.rsqrt(deg), 0.0)


def _tc_a_body(x_ref, w_ref, dp_ref, o_ref):
    dinv = _dinv_from(dp_ref)
    h = jnp.dot(x_ref[...], w_ref[...], preferred_element_type=jnp.float32)
    o_ref[...] = h * dinv


def _tc_b_body(sp_ref, dp_ref, b_ref, w_ref, o_ref):
    dinv = _dinv_from(dp_ref)
    sm = sp_ref[0] + sp_ref[1]
    z = jnp.maximum(sm * dinv + b_ref[...], 0.0)
    h = jnp.dot(z, w_ref[...], preferred_element_type=jnp.float32)
    o_ref[...] = h * dinv


def _tc_c_body(sp_ref, dp_ref, b2_ref, w_ref, b3_ref, o_ref):
    dinv = _dinv_from(dp_ref)
    sm = sp_ref[0] + sp_ref[1]
    z = jnp.maximum(sm * dinv + b2_ref[...], 0.0)
    o_ref[...] = (
        jnp.dot(z, w_ref[...], preferred_element_type=jnp.float32) + b3_ref[...]
    )


def _tc_a(xp, W1, degp):
    return pl.pallas_call(
        _tc_a_body,
        grid=(GRID,),
        in_specs=[
            pl.BlockSpec((ROWS, N_FEAT), lambda i: (i, 0)),
            pl.BlockSpec((N_FEAT, N_HID), lambda i: (0, 0)),
            pl.BlockSpec((2, ROWS, DEG_W), lambda i: (0, i, 0)),
        ],
        out_specs=pl.BlockSpec((ROWS, N_HID), lambda i: (i, 0)),
        out_shape=jax.ShapeDtypeStruct((N_PAD, N_HID), jnp.float32),
    )(xp, W1, degp)


def _tc_b(sp, degp, b1, W2):
    return pl.pallas_call(
        _tc_b_body,
        grid=(GRID,),
        in_specs=[
            pl.BlockSpec((2, ROWS, N_HID), lambda i: (0, i, 0)),
            pl.BlockSpec((2, ROWS, DEG_W), lambda i: (0, i, 0)),
            pl.BlockSpec((1, N_HID), lambda i: (0, 0)),
            pl.BlockSpec((N_HID, N_HID), lambda i: (0, 0)),
        ],
        out_specs=pl.BlockSpec((ROWS, N_HID), lambda i: (i, 0)),
        out_shape=jax.ShapeDtypeStruct((N_PAD, N_HID), jnp.float32),
    )(sp, degp, b1, W2)


def _tc_c(sp, degp, b2, W3, b3):
    return pl.pallas_call(
        _tc_c_body,
        grid=(GRID,),
        in_specs=[
            pl.BlockSpec((2, ROWS, N_HID), lambda i: (0, i, 0)),
            pl.BlockSpec((2, ROWS, DEG_W), lambda i: (0, i, 0)),
            pl.BlockSpec((1, N_HID), lambda i: (0, 0)),
            pl.BlockSpec((N_HID, N_CLASS), lambda i: (0, 0)),
            pl.BlockSpec((1, N_CLASS), lambda i: (0, 0)),
        ],
        out_specs=pl.BlockSpec((ROWS, N_CLASS), lambda i: (i, 0)),
        out_shape=jax.ShapeDtypeStruct((N_PAD, N_CLASS), jnp.float32),
    )(sp, degp, b2, W3, b3)


def kernel(x, edge_index, W1, b1, W2, b2, W3, b3):
    ei = edge_index.astype(jnp.int32)
    loop_idx = jnp.arange(N_NODES, dtype=jnp.int32)
    pad = jnp.full((E_PAD - N_EDGES_LOOP,), N_PAD - 1, jnp.int32)
    src = jnp.concatenate([ei[0], loop_idx, pad])
    dst = jnp.concatenate([ei[1], loop_idx, pad])
    xp = jnp.pad(x, ((0, N_PAD - N_NODES), (0, 0)))

    degp = _deg_kernel(dst)
    g1 = _tc_a(xp, W1, degp)
    s1 = _seg_sum_kernel(g1, src, dst)
    g2 = _tc_b(s1, degp, b1.reshape(1, -1), W2)
    s2 = _seg_sum_kernel(g2, src, dst)
    out = _tc_c(s2, degp, b2.reshape(1, -1), W3, b3.reshape(1, -1))
    return out[:N_NODES]


# speedup vs baseline: 1.5225x; 1.0005x over previous
import functools

import jax
import jax.numpy as jnp
from jax import lax
from jax.experimental import pallas as pl
from jax.experimental.pallas import tpu as pltpu
from jax.experimental.pallas import tpu_sc as plsc

N_NODES = 10000
N_PAD = 10240
N_FEAT = 128
N_HID = 128
N_CLASS = 64
N_EDGES = 320000
N_EDGES_LOOP = N_EDGES + N_NODES

CHUNK = 128
N_WORKERS = 32
CHUNKS_PER_W = 81
EDGES_PER_W = CHUNKS_PER_W * CHUNK
E_PAD = N_WORKERS * EDGES_PER_W
STRIPE = N_PAD // 16
DEG_W = 128

_mesh = plsc.VectorSubcoreMesh(core_axis_name="c", subcore_axis_name="s")


@functools.partial(
    pl.kernel,
    mesh=_mesh,
    out_type=jax.ShapeDtypeStruct((2, N_PAD, DEG_W), jnp.float32),
    scratch_types=[
        pltpu.VMEM((CHUNK,), jnp.int32),
        pltpu.VMEM((CHUNK, DEG_W), jnp.float32),
        pltpu.VMEM((16, DEG_W), jnp.float32),
        pltpu.VMEM_SHARED((N_PAD, DEG_W), jnp.float32),
    ],
)
def _deg_kernel(dst_hbm, out_hbm, dst_v, ones_v, zrow_v, d_sh):
    c = lax.axis_index("c")
    s = lax.axis_index("s")

    zero16 = jnp.zeros((16,), jnp.float32)
    one16 = jnp.ones((16,), jnp.float32)
    for r in range(16):
        for q in range(DEG_W // 16):
            zrow_v[r, pl.ds(q * 16, 16)] = zero16
    for r in range(CHUNK):
        for q in range(DEG_W // 16):
            ones_v[r, pl.ds(q * 16, 16)] = one16

    base_row = s * STRIPE
    def zero_body(i, carry):
        pltpu.sync_copy(zrow_v, d_sh.at[pl.ds(base_row + i * 16, 16)])
        return carry
    lax.fori_loop(0, STRIPE // 16, zero_body, 0)
    plsc.subcore_barrier()

    wid = c * 16 + s
    def edge_body(j, carry):
        off = (j * N_WORKERS + wid) * CHUNK
        pltpu.sync_copy(dst_hbm.at[pl.ds(off, CHUNK)], dst_v)
        pltpu.sync_copy(ones_v, d_sh.at[dst_v], add=True)
        return carry
    lax.fori_loop(0, CHUNKS_PER_W, edge_body, 0)
    plsc.subcore_barrier()

    pltpu.sync_copy(d_sh.at[pl.ds(base_row, STRIPE)],
                    out_hbm.at[c, pl.ds(base_row, STRIPE)])


@functools.partial(
    pl.kernel,
    mesh=_mesh,
    out_type=jax.ShapeDtypeStruct((2, N_PAD, N_FEAT), jnp.float32),
    scratch_types=[
        pltpu.VMEM((CHUNK,), jnp.int32),
        pltpu.VMEM((CHUNK,), jnp.int32),
        pltpu.VMEM((CHUNK, N_FEAT), jnp.float32),
        pltpu.VMEM((16, N_FEAT), jnp.float32),
        pltpu.VMEM_SHARED((N_PAD, N_FEAT), jnp.float32),
        pltpu.SemaphoreType.DMA,
    ],
)
def _seg_sum_kernel(g_hbm, src_hbm, dst_hbm, out_hbm,
                    src_v, dst_v, rows_v, zrow_v, s_sh, sem):
    c = lax.axis_index("c")
    s = lax.axis_index("s")

    zero16 = jnp.zeros((16,), jnp.float32)
    for r in range(16):
        for q in range(N_FEAT // 16):
            zrow_v[r, pl.ds(q * 16, 16)] = zero16

    base_row = s * STRIPE
    def zero_body(i, carry):
        pltpu.sync_copy(zrow_v, s_sh.at[pl.ds(base_row + i * 16, 16)])
        return carry
    lax.fori_loop(0, STRIPE // 16, zero_body, 0)
    plsc.subcore_barrier()

    wid = c * 16 + s
    def edge_body(j, carry):
        off = (j * N_WORKERS + wid) * CHUNK
        pltpu.sync_copy(src_hbm.at[pl.ds(off, CHUNK)], src_v)
        pltpu.sync_copy(dst_hbm.at[pl.ds(off, CHUNK)], dst_v)
        pltpu.async_copy(g_hbm.at[src_v], rows_v, sem).wait()
        pltpu.sync_copy(rows_v, s_sh.at[dst_v], add=True)
        return carry
    lax.fori_loop(0, CHUNKS_PER_W, edge_body, 0)
    plsc.subcore_barrier()

    pltpu.sync_copy(s_sh.at[pl.ds(base_row, STRIPE)],
                    out_hbm.at[c, pl.ds(base_row, STRIPE)])


ROWS = 1024
GRID = N_PAD // ROWS


def _dinv_from(dp_ref):
    deg = dp_ref[0, :, 0:1] + dp_ref[1, :, 0:1]
    return jnp.where(deg > 0.0, lax.rsqrt(deg), 0.0)


def _tc_mm_body(x_ref, w_ref, o_ref):
    o_ref[...] = jnp.dot(x_ref[...], w_ref[...],
                         preferred_element_type=jnp.float32)


def _tc_scale_body(h_ref, dp_ref, o_ref):
    o_ref[...] = h_ref[...] * _dinv_from(dp_ref)


def _tc_b_body(sp_ref, dp_ref, b_ref, w_ref, o_ref):
    dinv = _dinv_from(dp_ref)
    sm = sp_ref[0] + sp_ref[1]
    z = jnp.maximum(sm * dinv + b_ref[...], 0.0)
    h = jnp.dot(z, w_ref[...], preferred_element_type=jnp.float32)
    o_ref[...] = h * dinv


def _tc_c_body(sp_ref, dp_ref, b2_ref, w_ref, b3_ref, o_ref):
    dinv = _dinv_from(dp_ref)
    sm = sp_ref[0] + sp_ref[1]
    z = jnp.maximum(sm * dinv + b2_ref[...], 0.0)
    o_ref[...] = (
        jnp.dot(z, w_ref[...], preferred_element_type=jnp.float32) + b3_ref[...]
    )


def _tc_mm(xp, W1):
    return pl.pallas_call(
        _tc_mm_body,
        grid=(GRID,),
        in_specs=[
            pl.BlockSpec((ROWS, N_FEAT), lambda i: (i, 0)),
            pl.BlockSpec((N_FEAT, N_HID), lambda i: (0, 0)),
        ],
        out_specs=pl.BlockSpec((ROWS, N_HID), lambda i: (i, 0)),
        out_shape=jax.ShapeDtypeStruct((N_PAD, N_HID), jnp.float32),
    )(xp, W1)


def _tc_scale(h, degp):
    return pl.pallas_call(
        _tc_scale_body,
        grid=(GRID,),
        in_specs=[
            pl.BlockSpec((ROWS, N_HID), lambda i: (i, 0)),
            pl.BlockSpec((2, ROWS, DEG_W), lambda i: (0, i, 0)),
        ],
        out_specs=pl.BlockSpec((ROWS, N_HID), lambda i: (i, 0)),
        out_shape=jax.ShapeDtypeStruct((N_PAD, N_HID), jnp.float32),
    )(h, degp)


def _tc_b(sp, degp, b1, W2):
    return pl.pallas_call(
        _tc_b_body,
        grid=(GRID,),
        in_specs=[
            pl.BlockSpec((2, ROWS, N_HID), lambda i: (0, i, 0)),
            pl.BlockSpec((2, ROWS, DEG_W), lambda i: (0, i, 0)),
            pl.BlockSpec((1, N_HID), lambda i: (0, 0)),
            pl.BlockSpec((N_HID, N_HID), lambda i: (0, 0)),
        ],
        out_specs=pl.BlockSpec((ROWS, N_HID), lambda i: (i, 0)),
        out_shape=jax.ShapeDtypeStruct((N_PAD, N_HID), jnp.float32),
    )(sp, degp, b1, W2)


def _tc_c(sp, degp, b2, W3, b3):
    return pl.pallas_call(
        _tc_c_body,
        grid=(GRID,),
        in_specs=[
            pl.BlockSpec((2, ROWS, N_HID), lambda i: (0, i, 0)),
            pl.BlockSpec((2, ROWS, DEG_W), lambda i: (0, i, 0)),
            pl.BlockSpec((1, N_HID), lambda i: (0, 0)),
            pl.BlockSpec((N_HID, N_CLASS), lambda i: (0, 0)),
            pl.BlockSpec((1, N_CLASS), lambda i: (0, 0)),
        ],
        out_specs=pl.BlockSpec((ROWS, N_CLASS), lambda i: (i, 0)),
        out_shape=jax.ShapeDtypeStruct((N_PAD, N_CLASS), jnp.float32),
    )(sp, degp, b2, W3, b3)


def kernel(x, edge_index, W1, b1, W2, b2, W3, b3):
    ei = edge_index.astype(jnp.int32)
    loop_idx = jnp.arange(N_NODES, dtype=jnp.int32)
    pad = jnp.full((E_PAD - N_EDGES_LOOP,), N_PAD - 1, jnp.int32)
    src = jnp.concatenate([ei[0], loop_idx, pad])
    dst = jnp.concatenate([ei[1], loop_idx, pad])
    xp = jnp.pad(x, ((0, N_PAD - N_NODES), (0, 0)))

    h1 = _tc_mm(xp, W1)
    degp = _deg_kernel(dst)
    g1 = _tc_scale(h1, degp)
    s1 = _seg_sum_kernel(g1, src, dst)
    g2 = _tc_b(s1, degp, b1.reshape(1, -1), W2)
    s2 = _seg_sum_kernel(g2, src, dst)
    out = _tc_c(s2, degp, b2.reshape(1, -1), W3, b3.reshape(1, -1))
    return out[:N_NODES]
